# Initial kernel scaffold; baseline (speedup 1.0000x reference)
import jax
import jax.numpy as jnp
from jax.experimental import pallas as pl


def kernel(x, table):
    raise NotImplementedError("write your pallas kernel here")


# speedup vs baseline: 2.9066x; 2.9066x over previous
import functools
import jax
import jax.numpy as jnp
from jax import lax
from jax.experimental import pallas as pl
from jax.experimental.pallas import tpu as pltpu
from jax.experimental.pallas import tpu_sc as plsc

NUM_DEVICE_TYPES = 100000
EMBED_DIM = 64
BATCH = 4096
HIST_LEN = 50
SCALE = 8.0

NC = 2
NS = 16
NW = NC * NS
TOTAL = BATCH * HIST_LEN
PER_W = TOTAL // NW
CHUNK = 128
NCHUNK = PER_W // CHUNK


@functools.partial(
    pl.kernel,
    mesh=plsc.VectorSubcoreMesh(core_axis_name="c", subcore_axis_name="s"),
    out_type=jax.ShapeDtypeStruct((TOTAL, EMBED_DIM), jnp.float32),
    scratch_types=[
        pltpu.VMEM((NCHUNK, CHUNK), jnp.int32),
        pltpu.VMEM((CHUNK, EMBED_DIM), jnp.float32),
        pltpu.SemaphoreType.DMA,
    ],
    compiler_params=pltpu.CompilerParams(use_tc_tiling_on_sc=False),
)
def _embed_gather(table_hbm, idx_hbm, out_hbm, idx_v, rows_v, sem):
    wid = lax.axis_index("s") * NC + lax.axis_index("c")
    base = wid * PER_W
    pltpu.sync_copy(idx_hbm.at[wid], idx_v)

    def chunk_body(j, carry):
        pltpu.async_copy(table_hbm.at[idx_v.at[j]], rows_v, sem).wait()

        def row_body(r, c2):
            for c in range(EMBED_DIM // 16):
                sl = pl.ds(c * 16, 16)
                rows_v[r, sl] = rows_v[r, sl] * SCALE
            return c2

        lax.fori_loop(0, CHUNK, row_body, 0)
        pltpu.sync_copy(rows_v, out_hbm.at[pl.ds(base + j * CHUNK, CHUNK)])
        return carry

    lax.fori_loop(0, NCHUNK, chunk_body, 0)


def kernel(x, table):
    idx = x.astype(jnp.int32).reshape(NW, NCHUNK, CHUNK)
    out = _embed_gather(table, idx)
    return out.reshape(BATCH, HIST_LEN, EMBED_DIM)


# speedup vs baseline: 3.6586x; 1.2588x over previous
import functools
import jax
import jax.numpy as jnp
from jax import lax
from jax.experimental import pallas as pl
from jax.experimental.pallas import tpu as pltpu
from jax.experimental.pallas import tpu_sc as plsc

NUM_DEVICE_TYPES = 100000
EMBED_DIM = 64
BATCH = 4096
HIST_LEN = 50
SCALE = 8.0

NC = 2
NS = 16
NW = NC * NS
TOTAL = BATCH * HIST_LEN
PER_W = TOTAL // NW
CHUNK = 128
NCHUNK = PER_W // CHUNK
NBUF = 5
LOOKAHEAD = NBUF - 1


@functools.partial(
    pl.kernel,
    mesh=plsc.VectorSubcoreMesh(core_axis_name="c", subcore_axis_name="s"),
    out_type=jax.ShapeDtypeStruct((TOTAL, EMBED_DIM), jnp.float32),
    scratch_types=[
        pltpu.VMEM((NCHUNK, CHUNK), jnp.int32),
        pltpu.VMEM((NBUF, CHUNK, EMBED_DIM), jnp.float32),
        pltpu.SemaphoreType.DMA((NBUF,)),
        pltpu.SemaphoreType.DMA((NBUF,)),
    ],
    compiler_params=pltpu.CompilerParams(use_tc_tiling_on_sc=False),
)
def _embed_gather(table_hbm, idx_hbm, out_hbm, idx_v, rows_v, gsem, osem):
    wid = lax.axis_index("s") * NC + lax.axis_index("c")
    base = wid * PER_W
    pltpu.sync_copy(idx_hbm.at[wid], idx_v)

    def gather(j, b):
        return pltpu.make_async_copy(
            table_hbm.at[idx_v.at[j]], rows_v.at[b], gsem.at[b])

    def writeback(j, b):
        return pltpu.make_async_copy(
            rows_v.at[b], out_hbm.at[pl.ds(base + j * CHUNK, CHUNK)],
            osem.at[b])

    for b in range(LOOKAHEAD):
        gather(b, b).start()

    def outer(g, carry):
        for b in range(NBUF):
            j = g * NBUF + b
            bb = (b + LOOKAHEAD) % NBUF
            jj = j + LOOKAHEAD

            @pl.when(jj < NCHUNK)
            def _():
                @pl.when(j >= 1)
                def _():
                    writeback(j - 1, bb).wait()

                gather(jj, bb).start()

            gather(j, b).wait()

            rv = rows_v.at[b]

            def scale_rows(r0, c2):
                for dr in range(4):
                    r = r0 * 4 + dr
                    for c in range(EMBED_DIM // 16):
                        sl = pl.ds(c * 16, 16)
                        rv[r, sl] = rv[r, sl] * SCALE
                return c2

            lax.fori_loop(0, CHUNK // 4, scale_rows, 0)
            writeback(j, b).start()
        return carry

    lax.fori_loop(0, NCHUNK // NBUF, outer, 0)

    for b in range(NBUF):
        writeback(NCHUNK - NBUF + b, b).wait()


def kernel(x, table):
    idx = x.astype(jnp.int32).reshape(NW, NCHUNK, CHUNK)
    out = _embed_gather(table, idx)
    return out.reshape(BATCH, HIST_LEN, EMBED_DIM)
